# baseline (speedup 1.0000x reference)
import jax
import jax.numpy as jnp
from jax import lax
from jax.experimental import pallas as pl
from jax.experimental.pallas import tpu as pltpu
from jax.experimental.pallas import tpu_sc as plsc

_N = 100000
_E = 1600000
_HALF = 16
_NT = 16
_EPT = _E // _NT
_CH = 400
_NCH = _EPT // _CH
_RPT = _N // _NT
_ECH = 250
_NECH = _RPT // _ECH


def _sc_body(w4, e1, e2, pos_o, neg_o, h_o, acc, src_v, dst_v, rows_v,
             buf_v, zero_v):
    c = lax.axis_index("c")
    s = lax.axis_index("s")
    row0 = s * _RPT

    def _zset(i, _):
        zero_v[i, :] = jnp.zeros((_HALF,), jnp.float32)
        return 0
    lax.fori_loop(0, _ECH, _zset, 0)

    def _zacc(k, _):
        r0 = pl.multiple_of(row0 + k * _ECH, 8)
        pltpu.sync_copy(zero_v, acc.at[pl.ds(r0, _ECH)])
        return 0
    lax.fori_loop(0, _NECH, _zacc, 0)
    plsc.subcore_barrier()

    def _accumulate(e, table, do_xform):
        base = s * _EPT

        def _step(k, _):
            off = pl.multiple_of(base + k * _CH, 8)
            pltpu.sync_copy(e.at[0, pl.ds(off, _CH)], src_v)
            pltpu.sync_copy(e.at[1, pl.ds(off, _CH)], dst_v)
            if do_xform:
                def body(i, _):
                    v = src_v[pl.ds(i * 16, 16)]
                    src_v[pl.ds(i * 16, 16)] = v + v + c
                    return 0
                lax.fori_loop(0, _CH // 16, body, 0)
            pltpu.sync_copy(table.at[src_v], rows_v)
            pltpu.sync_copy(rows_v, acc.at[dst_v], add=True)
            return 0
        lax.fori_loop(0, _NCH, _step, 0)
        plsc.subcore_barrier()

    def _evacuate(out_at, do_relu):
        def _step(k, _):
            r0 = pl.multiple_of(row0 + k * _ECH, 8)
            pltpu.sync_copy(acc.at[pl.ds(r0, _ECH)], buf_v)
            if do_relu:
                def _relu_row(i, _):
                    buf_v[i, :] = jnp.maximum(buf_v[i, :], 0.0)
                    return 0
                lax.fori_loop(0, _ECH, _relu_row, 0)
            pltpu.sync_copy(buf_v, out_at(r0))
            pltpu.sync_copy(zero_v, acc.at[pl.ds(r0, _ECH)])
            return 0
        lax.fori_loop(0, _NECH, _step, 0)
        plsc.subcore_barrier()

    def _h_at(r0):
        return h_o.at[c].at[pl.ds(r0, _ECH)]

    for e, out in ((e1, pos_o), (e2, neg_o)):
        def _out_at(r0, out=out):
            return out.at[pl.ds(r0, _ECH), c]

        _accumulate(e, w4, True)
        _evacuate(_h_at, True)
        _accumulate(e, h_o.at[c], False)
        _evacuate(_out_at, False)


def kernel(inputs, edge_index1, edge_index2, w):
    del inputs
    w4 = w.reshape(2 * _N, _HALF)
    mesh = plsc.VectorSubcoreMesh(core_axis_name="c", subcore_axis_name="s")
    f = pl.kernel(
        _sc_body,
        out_type=[
            jax.ShapeDtypeStruct((_N, 2, _HALF), jnp.float32),
            jax.ShapeDtypeStruct((_N, 2, _HALF), jnp.float32),
            jax.ShapeDtypeStruct((2, _N, _HALF), jnp.float32),
        ],
        mesh=mesh,
        scratch_types=[
            pltpu.VMEM_SHARED((_N, _HALF), jnp.float32),
            pltpu.VMEM((_CH,), jnp.int32),
            pltpu.VMEM((_CH,), jnp.int32),
            pltpu.VMEM((_CH, _HALF), jnp.float32),
            pltpu.VMEM((_ECH, _HALF), jnp.float32),
            pltpu.VMEM((_ECH, _HALF), jnp.float32),
        ],
        compiler_params=pltpu.CompilerParams(use_tc_tiling_on_sc=False),
    )
    pos4, neg4, _ = f(w4, edge_index1, edge_index2)
    return pos4.reshape(_N, 32), neg4.reshape(_N, 32)


# speedup vs baseline: 2.4102x; 2.4102x over previous
import jax
import jax.numpy as jnp
from jax import lax
from jax.experimental import pallas as pl
from jax.experimental.pallas import tpu as pltpu
from jax.experimental.pallas import tpu_sc as plsc

_N = 100000
_E = 1600000
_HALF = 16
_NT = 16
_EPT = _E // _NT
_CH = 400
_NCH = _EPT // _CH
_RPT = _N // _NT
_ECH = 250
_NECH = _RPT // _ECH


def _sc_body(w4, e1, e2, pos_o, neg_o, h_o, acc,
             si0, si1, si2, si3, di0, di1, di2, di3, rw0, rw1,
             buf_v, zero_v,
             smi0, smi1, smi2, smi3, smg0, smg1, sms0, sms1):
    c = lax.axis_index("c")
    s = lax.axis_index("s")
    row0 = s * _RPT
    srci = (si0, si1, si2, si3)
    dsti = (di0, di1, di2, di3)
    rows = (rw0, rw1)
    semi = (smi0, smi1, smi2, smi3)
    semg = (smg0, smg1)
    sems = (sms0, sms1)

    def _zset(i, _):
        zero_v[i, :] = jnp.zeros((_HALF,), jnp.float32)
        return 0
    lax.fori_loop(0, _ECH, _zset, 0)

    def _zacc(k, _):
        r0 = pl.multiple_of(row0 + k * _ECH, 8)
        pltpu.sync_copy(zero_v, acc.at[pl.ds(r0, _ECH)])
        return 0
    lax.fori_loop(0, _NECH, _zacc, 0)
    plsc.subcore_barrier()

    def _accumulate(e, table, do_xform):
        base = s * _EPT

        def _off(k):
            return pl.multiple_of(base + k * _CH, 8)

        def idx_start(k, j):
            off = _off(k)
            pltpu.async_copy(e.at[0, pl.ds(off, _CH)], srci[j], semi[j])
            pltpu.async_copy(e.at[1, pl.ds(off, _CH)], dsti[j], semi[j])

        def idx_wait(k, j):
            off = _off(k)
            pltpu.make_async_copy(e.at[0, pl.ds(off, _CH)], srci[j],
                                  semi[j]).wait()
            pltpu.make_async_copy(e.at[1, pl.ds(off, _CH)], dsti[j],
                                  semi[j]).wait()

        def xform(j):
            if not do_xform:
                return
            sl = srci[j]

            def body(i, _):
                v = sl[pl.ds(i * 16, 16)]
                sl[pl.ds(i * 16, 16)] = v + v + c
                return 0
            lax.fori_loop(0, _CH // 16, body, 0)

        def gather_start(j, b):
            pltpu.async_copy(table.at[srci[j]], rows[b], semg[b])

        def gather_wait(j, b):
            pltpu.make_async_copy(table.at[srci[j]], rows[b], semg[b]).wait()

        def scat_start(j, b):
            pltpu.async_copy(rows[b], acc.at[dsti[j]], sems[b], add=True)

        def scat_wait(j, b):
            pltpu.make_async_copy(rows[b], acc.at[dsti[j]], sems[b]).wait()

        idx_start(0, 0)
        idx_start(1, 1)
        idx_wait(0, 0)
        xform(0)
        gather_start(0, 0)

        def group(g, _):
            for j in range(4):
                k = g * 4 + j
                b = j % 2
                ob = 1 - b

                @pl.when(k >= 1)
                def _(j=j, ob=ob):
                    scat_wait((j + 3) % 4, ob)
                if j % 2 == 0:
                    @pl.when(k + 2 < _NCH)
                    def _(k=k, j=j):
                        idx_start(k + 2, (j + 2) % 4)

                    @pl.when(k + 3 < _NCH)
                    def _(k=k, j=j):
                        idx_start(k + 3, (j + 3) % 4)

                @pl.when(k + 1 < _NCH)
                def _(k=k, j=j, ob=ob):
                    idx_wait(k + 1, (j + 1) % 4)
                    xform((j + 1) % 4)
                    gather_start((j + 1) % 4, ob)
                gather_wait(j, b)
                scat_start(j, b)
            return 0
        lax.fori_loop(0, _NCH // 4, group, 0)

        scat_wait(3, 1)
        idx_wait(_NCH - 1, 1)
        xform(1)
        gather_start(1, 1)
        gather_wait(0, 0)
        scat_start(0, 0)
        scat_wait(0, 0)
        gather_wait(1, 1)
        scat_start(1, 1)
        scat_wait(1, 1)
        plsc.subcore_barrier()

    def _evacuate(out_at, do_relu):
        def _step(k, _):
            r0 = pl.multiple_of(row0 + k * _ECH, 8)
            pltpu.sync_copy(acc.at[pl.ds(r0, _ECH)], buf_v)
            if do_relu:
                def _relu_row(i, _):
                    buf_v[i, :] = jnp.maximum(buf_v[i, :], 0.0)
                    return 0
                lax.fori_loop(0, _ECH, _relu_row, 0)
            pltpu.sync_copy(buf_v, out_at(r0))
            pltpu.sync_copy(zero_v, acc.at[pl.ds(r0, _ECH)])
            return 0
        lax.fori_loop(0, _NECH, _step, 0)
        plsc.subcore_barrier()

    def _h_at(r0):
        return h_o.at[c].at[pl.ds(r0, _ECH)]

    for e, out in ((e1, pos_o), (e2, neg_o)):
        def _out_at(r0, out=out):
            return out.at[pl.ds(r0, _ECH), c]

        _accumulate(e, w4, True)
        _evacuate(_h_at, True)
        _accumulate(e, h_o.at[c], False)
        _evacuate(_out_at, False)


def kernel(inputs, edge_index1, edge_index2, w):
    del inputs
    w4 = w.reshape(2 * _N, _HALF)
    mesh = plsc.VectorSubcoreMesh(core_axis_name="c", subcore_axis_name="s")
    f = pl.kernel(
        _sc_body,
        out_type=[
            jax.ShapeDtypeStruct((_N, 2, _HALF), jnp.float32),
            jax.ShapeDtypeStruct((_N, 2, _HALF), jnp.float32),
            jax.ShapeDtypeStruct((2, _N, _HALF), jnp.float32),
        ],
        mesh=mesh,
        scratch_types=[
            pltpu.VMEM_SHARED((_N, _HALF), jnp.float32),
            pltpu.VMEM((_CH,), jnp.int32),
            pltpu.VMEM((_CH,), jnp.int32),
            pltpu.VMEM((_CH,), jnp.int32),
            pltpu.VMEM((_CH,), jnp.int32),
            pltpu.VMEM((_CH,), jnp.int32),
            pltpu.VMEM((_CH,), jnp.int32),
            pltpu.VMEM((_CH,), jnp.int32),
            pltpu.VMEM((_CH,), jnp.int32),
            pltpu.VMEM((_CH, _HALF), jnp.float32),
            pltpu.VMEM((_CH, _HALF), jnp.float32),
            pltpu.VMEM((_ECH, _HALF), jnp.float32),
            pltpu.VMEM((_ECH, _HALF), jnp.float32),
            pltpu.SemaphoreType.DMA,
            pltpu.SemaphoreType.DMA,
            pltpu.SemaphoreType.DMA,
            pltpu.SemaphoreType.DMA,
            pltpu.SemaphoreType.DMA,
            pltpu.SemaphoreType.DMA,
            pltpu.SemaphoreType.DMA,
            pltpu.SemaphoreType.DMA,
        ],
        compiler_params=pltpu.CompilerParams(use_tc_tiling_on_sc=False),
    )
    pos4, neg4, _ = f(w4, edge_index1, edge_index2)
    return pos4.reshape(_N, 32), neg4.reshape(_N, 32)
